# baseline (speedup 1.0000x reference)
import jax
import jax.numpy as jnp
from jax import lax
from jax.experimental import pallas as pl
from jax.experimental.pallas import tpu as pltpu
from jax.experimental.pallas import tpu_sc as plsc

B = 8
N = 8192
Q = 16
G = 32
G3 = G * G * G
NW = 32
PPW = N * B // NW
HALF = PPW // 2
SUBH = HALF // 16


def _rsqrt(v):
    i = plsc.bitcast(v, jnp.int32)
    i = 0x5F3759DF - (i >> 1)
    y = plsc.bitcast(i, jnp.float32)
    for _ in range(4):
        y = y * (1.5 - 0.5 * v * y * y)
    return y


def _sc_body(pts_hbm, cp_hbm, vox_hbm, quads_hbm, out_hbm,
             px, py, pz, qbuf, coef_v, sumv,
             idxh, gxh, gyh, gzh, gvh,
             rbx, rby, rbz, accv, shared, sem):
    cid = lax.axis_index("c")
    sid = lax.axis_index("s")
    wid = cid * 16 + sid
    b = wid // 4
    p0 = (wid % 4) * PPW

    pltpu.sync_copy(pts_hbm.at[0 * B + b, pl.ds(p0, PPW)], px)
    pltpu.sync_copy(pts_hbm.at[1 * B + b, pl.ds(p0, PPW)], py)
    pltpu.sync_copy(pts_hbm.at[2 * B + b, pl.ds(p0, PPW)], pz)
    pltpu.sync_copy(quads_hbm.at[pl.ds(b * Q * 4, Q * 4)], qbuf)

    lane = lax.iota(jnp.int32, 16)

    def mean_body(i, carry):
        sx, sy, sz = carry
        off = pl.multiple_of(i * 16, 16)
        return (sx + px[pl.ds(off, 16)],
                sy + py[pl.ds(off, 16)],
                sz + pz[pl.ds(off, 16)])

    zero16 = jnp.zeros((16,), jnp.float32)
    sx, sy, sz = lax.fori_loop(0, PPW // 16, mean_body, (zero16, zero16, zero16))
    sumv[pl.ds(0, 16)] = sx
    sumv[pl.ds(16, 16)] = sy
    sumv[pl.ds(32, 16)] = sz
    pltpu.sync_copy(sumv, shared.at[sid])
    plsc.subcore_barrier()
    w0 = (b % 4) * 4
    pltpu.sync_copy(shared.at[w0], sumv)
    s0x, s0y, s0z = sumv[pl.ds(0, 16)], sumv[pl.ds(16, 16)], sumv[pl.ds(32, 16)]
    pltpu.sync_copy(shared.at[w0 + 1], sumv)
    s1x, s1y, s1z = sumv[pl.ds(0, 16)], sumv[pl.ds(16, 16)], sumv[pl.ds(32, 16)]
    pltpu.sync_copy(shared.at[w0 + 2], sumv)
    s2x, s2y, s2z = sumv[pl.ds(0, 16)], sumv[pl.ds(16, 16)], sumv[pl.ds(32, 16)]
    pltpu.sync_copy(shared.at[w0 + 3], sumv)
    s3x, s3y, s3z = sumv[pl.ds(0, 16)], sumv[pl.ds(16, 16)], sumv[pl.ds(32, 16)]
    inv_n = 1.0 / N
    midx = jnp.sum((s0x + s1x) + (s2x + s3x)) * inv_n
    midy = jnp.sum((s0y + s1y) + (s2y + s3y)) * inv_n
    midz = jnp.sum((s0z + s1z) + (s2z + s3z)) * inv_n

    qx = plsc.load_gather(qbuf, [lane * 4 + 1])
    qy = plsc.load_gather(qbuf, [lane * 4 + 2])
    qz = plsc.load_gather(qbuf, [lane * 4 + 3])
    rinv = _rsqrt(qx * qx + qy * qy + qz * qz)
    w = jnp.full((16,), 0.707, jnp.float32)
    x = 0.707 * (qx * rinv)
    y = 0.707 * (qy * rinv)
    z = 0.707 * (qz * rinv)
    m00 = w * w + x * x - y * y - z * z
    m01 = 2.0 * (x * y - w * z)
    m02 = 2.0 * (x * z + w * y)
    m10 = 2.0 * (x * y + w * z)
    m11 = w * w - x * x + y * y - z * z
    m12 = 2.0 * (y * z - w * x)
    m20 = 2.0 * (x * z - w * y)
    m21 = 2.0 * (y * z + w * x)
    m22 = w * w - x * x - y * y + z * z
    t0 = -(m00 * midx + m01 * midy + m02 * midz)
    t1 = -(m10 * midx + m11 * midy + m12 * midz)
    t2 = -(m20 * midx + m21 * midy + m22 * midz)
    lane16 = lane * 16
    for k, vec in enumerate((m00, m01, m02, m10, m11, m12, m20, m21, m22,
                             t0, t1, t2)):
        plsc.store_scatter(coef_v, [lane16 + k], vec)

    wq = (wid % 4) * 4

    def phase_a(h):
        def body(i, _):
            off = pl.multiple_of(h * HALF + i * 16, 16)
            qb = wq + (h * 2 + i // 32)
            crow = coef_v[pl.ds(pl.multiple_of(qb * 16, 16), 16)]
            vx = px[pl.ds(off, 16)]
            vy = py[pl.ds(off, 16)]
            vz = pz[pl.ds(off, 16)]
            rx = crow[0] * vx + crow[1] * vy + crow[2] * vz + crow[9]
            ry = crow[3] * vx + crow[4] * vy + crow[5] * vz + crow[10]
            rz = crow[6] * vx + crow[7] * vy + crow[8] * vz + crow[11]

            def vceil(t):
                ti = t.astype(jnp.int32)
                tf = ti.astype(jnp.float32)
                return ti + jnp.where(t > tf, 1, 0).astype(jnp.int32)

            ix = vceil((rx + 0.5) * G - 0.5)
            iy = vceil((ry + 0.5) * G - 0.5)
            iz = vceil((rz + 0.5) * G - 0.5)
            ind = ix * (G * G) + iy * G + iz
            ind = jnp.minimum(jnp.maximum(ind, 0), G3 - 1)
            rbx[pl.ds(off, 16)] = rx
            rby[pl.ds(off, 16)] = ry
            rbz[pl.ds(off, 16)] = rz
            soff = pl.multiple_of(i * 16, 16)
            idxh[h][pl.ds(soff, 16)] = ind
            return 0

        lax.fori_loop(0, SUBH, body, 0)

    def fire(h):
        pltpu.async_copy(cp_hbm.at[0 * B + b].at[idxh[h]], gxh[h], sem)
        pltpu.async_copy(cp_hbm.at[1 * B + b].at[idxh[h]], gyh[h], sem)
        pltpu.async_copy(cp_hbm.at[2 * B + b].at[idxh[h]], gzh[h], sem)
        pltpu.async_copy(vox_hbm.at[b].at[idxh[h]], gvh[h], sem)

    def drain(h):
        pltpu.make_async_copy(cp_hbm.at[0 * B + b].at[idxh[h]], gxh[h], sem).wait()
        pltpu.make_async_copy(cp_hbm.at[1 * B + b].at[idxh[h]], gyh[h], sem).wait()
        pltpu.make_async_copy(cp_hbm.at[2 * B + b].at[idxh[h]], gzh[h], sem).wait()
        pltpu.make_async_copy(vox_hbm.at[b].at[idxh[h]], gvh[h], sem).wait()

    def phase_c(h, acc):
        def body(i, acc):
            off = pl.multiple_of(h * HALF + i * 16, 16)
            soff = pl.multiple_of(i * 16, 16)
            m = 1.0 - gvh[h][pl.ds(soff, 16)]
            dx = (rbx[pl.ds(off, 16)] - gxh[h][pl.ds(soff, 16)]) * m
            dy = (rby[pl.ds(off, 16)] - gyh[h][pl.ds(soff, 16)]) * m
            dz = (rbz[pl.ds(off, 16)] - gzh[h][pl.ds(soff, 16)]) * m
            return acc + (dx * dx + dy * dy + dz * dz)

        return lax.fori_loop(0, SUBH, body, acc)

    phase_a(0)
    fire(0)
    phase_a(1)
    fire(1)
    drain(0)
    acc = phase_c(0, jnp.zeros((16,), jnp.float32))
    drain(1)
    acc = phase_c(1, acc)

    accv[pl.ds(0, 16)] = acc
    for r in range(1, 8):
        accv[pl.ds(r * 16, 16)] = zero16
    pltpu.sync_copy(accv, out_hbm.at[wid])


def _tc_reduce_body(x_ref, o_ref):
    o_ref[0, 0] = jnp.sum(x_ref[...]) * (1.0 / (B * N))


def kernel(voxel, points, closest_points, quads):
    pts_cm = points.transpose(2, 0, 1).reshape(3 * B, N)
    cp_cm = closest_points.transpose(2, 0, 1).reshape(3 * B, G3)
    vox_2d = voxel.reshape(B, G3)
    quads_flat = quads.reshape(B * Q * 4)

    mesh = plsc.VectorSubcoreMesh(core_axis_name="c", subcore_axis_name="s")
    partials = pl.kernel(
        _sc_body,
        out_type=jax.ShapeDtypeStruct((NW, 128), jnp.float32),
        mesh=mesh,
        compiler_params=pltpu.CompilerParams(
            use_tc_tiling_on_sc=False, needs_layout_passes=False),
        scratch_types=[
            pltpu.VMEM((PPW,), jnp.float32),
            pltpu.VMEM((PPW,), jnp.float32),
            pltpu.VMEM((PPW,), jnp.float32),
            pltpu.VMEM((Q * 4,), jnp.float32),
            pltpu.VMEM((Q * 16,), jnp.float32),
            pltpu.VMEM((48,), jnp.float32),
            [pltpu.VMEM((HALF,), jnp.int32), pltpu.VMEM((HALF,), jnp.int32)],
            [pltpu.VMEM((HALF,), jnp.float32), pltpu.VMEM((HALF,), jnp.float32)],
            [pltpu.VMEM((HALF,), jnp.float32), pltpu.VMEM((HALF,), jnp.float32)],
            [pltpu.VMEM((HALF,), jnp.float32), pltpu.VMEM((HALF,), jnp.float32)],
            [pltpu.VMEM((HALF,), jnp.float32), pltpu.VMEM((HALF,), jnp.float32)],
            pltpu.VMEM((PPW,), jnp.float32),
            pltpu.VMEM((PPW,), jnp.float32),
            pltpu.VMEM((PPW,), jnp.float32),
            pltpu.VMEM((128,), jnp.float32),
            pltpu.VMEM_SHARED((16, 48), jnp.float32),
            pltpu.SemaphoreType.DMA,
        ],
    )(pts_cm, cp_cm, vox_2d, quads_flat)

    total = pl.pallas_call(
        _tc_reduce_body,
        out_shape=jax.ShapeDtypeStruct((1, 1), jnp.float32),
        out_specs=pl.BlockSpec(memory_space=pltpu.SMEM),
    )(partials)
    return total[0, 0]


# speedup vs baseline: 1.0925x; 1.0925x over previous
import jax
import jax.numpy as jnp
from jax import lax
from jax.experimental import pallas as pl
from jax.experimental.pallas import tpu as pltpu
from jax.experimental.pallas import tpu_sc as plsc

B = 8
N = 8192
Q = 16
G = 32
G3 = G * G * G
NW = 32
PPW = N * B // NW
CHUNK = 128
NCHUNK = PPW // CHUNK
SUB = CHUNK // 16


def _rsqrt(v):
    i = plsc.bitcast(v, jnp.int32)
    i = 0x5F3759DF - (i >> 1)
    y = plsc.bitcast(i, jnp.float32)
    for _ in range(4):
        y = y * (1.5 - 0.5 * v * y * y)
    return y


def _sc_body(pts_hbm, cp_hbm, vox_hbm, quads_hbm, out_hbm,
             px, py, pz, qbuf, coef_v, sumv,
             idxv, gxh, gyh, gzh, gvh,
             rbx, rby, rbz, accv, shared, sem):
    cid = lax.axis_index("c")
    sid = lax.axis_index("s")
    wid = cid * 16 + sid
    b = wid // 4
    p0 = (wid % 4) * PPW

    pltpu.sync_copy(pts_hbm.at[0, b, pl.ds(p0, PPW)], px)
    pltpu.sync_copy(pts_hbm.at[1, b, pl.ds(p0, PPW)], py)
    pltpu.sync_copy(pts_hbm.at[2, b, pl.ds(p0, PPW)], pz)
    pltpu.sync_copy(quads_hbm.at[pl.ds(b * Q * 4, Q * 4)], qbuf)

    lane = lax.iota(jnp.int32, 16)

    def mean_body(i, carry):
        sx, sy, sz = carry
        off = pl.multiple_of(i * 16, 16)
        return (sx + px[pl.ds(off, 16)],
                sy + py[pl.ds(off, 16)],
                sz + pz[pl.ds(off, 16)])

    zero16 = jnp.zeros((16,), jnp.float32)
    sx, sy, sz = lax.fori_loop(0, PPW // 16, mean_body, (zero16, zero16, zero16))
    sumv[pl.ds(0, 16)] = sx
    sumv[pl.ds(16, 16)] = sy
    sumv[pl.ds(32, 16)] = sz
    pltpu.sync_copy(sumv, shared.at[sid])
    plsc.subcore_barrier()
    w0 = (b % 4) * 4
    pltpu.sync_copy(shared.at[w0], sumv)
    s0x, s0y, s0z = sumv[pl.ds(0, 16)], sumv[pl.ds(16, 16)], sumv[pl.ds(32, 16)]
    pltpu.sync_copy(shared.at[w0 + 1], sumv)
    s1x, s1y, s1z = sumv[pl.ds(0, 16)], sumv[pl.ds(16, 16)], sumv[pl.ds(32, 16)]
    pltpu.sync_copy(shared.at[w0 + 2], sumv)
    s2x, s2y, s2z = sumv[pl.ds(0, 16)], sumv[pl.ds(16, 16)], sumv[pl.ds(32, 16)]
    pltpu.sync_copy(shared.at[w0 + 3], sumv)
    s3x, s3y, s3z = sumv[pl.ds(0, 16)], sumv[pl.ds(16, 16)], sumv[pl.ds(32, 16)]
    inv_n = 1.0 / N
    midx = jnp.sum((s0x + s1x) + (s2x + s3x)) * inv_n
    midy = jnp.sum((s0y + s1y) + (s2y + s3y)) * inv_n
    midz = jnp.sum((s0z + s1z) + (s2z + s3z)) * inv_n

    qx = plsc.load_gather(qbuf, [lane * 4 + 1])
    qy = plsc.load_gather(qbuf, [lane * 4 + 2])
    qz = plsc.load_gather(qbuf, [lane * 4 + 3])
    rinv = _rsqrt(qx * qx + qy * qy + qz * qz)
    w = jnp.full((16,), 0.707, jnp.float32)
    x = 0.707 * (qx * rinv)
    y = 0.707 * (qy * rinv)
    z = 0.707 * (qz * rinv)
    m00 = w * w + x * x - y * y - z * z
    m01 = 2.0 * (x * y - w * z)
    m02 = 2.0 * (x * z + w * y)
    m10 = 2.0 * (x * y + w * z)
    m11 = w * w - x * x + y * y - z * z
    m12 = 2.0 * (y * z - w * x)
    m20 = 2.0 * (x * z - w * y)
    m21 = 2.0 * (y * z + w * x)
    m22 = w * w - x * x - y * y + z * z
    t0 = -(m00 * midx + m01 * midy + m02 * midz)
    t1 = -(m10 * midx + m11 * midy + m12 * midz)
    t2 = -(m20 * midx + m21 * midy + m22 * midz)
    lane16 = lane * 16
    for k, vec in enumerate((m00, m01, m02, m10, m11, m12, m20, m21, m22,
                             t0, t1, t2)):
        plsc.store_scatter(coef_v, [lane16 + k], vec)

    wq = (wid % 4) * 4

    def chunk_fire(c, _):
        qb = wq + c // 4
        crow = coef_v[pl.ds(pl.multiple_of(qb * 16, 16), 16)]
        c00 = crow[0]
        c01 = crow[1]
        c02 = crow[2]
        c10 = crow[3]
        c11 = crow[4]
        c12 = crow[5]
        c20 = crow[6]
        c21 = crow[7]
        c22 = crow[8]
        d0 = crow[9]
        d1 = crow[10]
        d2 = crow[11]

        def phase_a(i, _):
            off = pl.multiple_of(c * CHUNK + i * 16, 16)
            vx = px[pl.ds(off, 16)]
            vy = py[pl.ds(off, 16)]
            vz = pz[pl.ds(off, 16)]
            rx = c00 * vx + c01 * vy + c02 * vz + d0
            ry = c10 * vx + c11 * vy + c12 * vz + d1
            rz = c20 * vx + c21 * vy + c22 * vz + d2

            def vceil(t):
                ti = t.astype(jnp.int32)
                tf = ti.astype(jnp.float32)
                return ti + jnp.where(t > tf, 1, 0).astype(jnp.int32)

            ix = vceil((rx + 0.5) * G - 0.5)
            iy = vceil((ry + 0.5) * G - 0.5)
            iz = vceil((rz + 0.5) * G - 0.5)
            ind = ix * (G * G) + iy * G + iz
            ind = jnp.minimum(jnp.maximum(ind, 0), G3 - 1)
            rbx[pl.ds(off, 16)] = rx
            rby[pl.ds(off, 16)] = ry
            rbz[pl.ds(off, 16)] = rz
            idxv[c, pl.ds(pl.multiple_of(i * 16, 16), 16)] = ind
            return 0

        lax.fori_loop(0, SUB, phase_a, 0)
        pltpu.async_copy(cp_hbm.at[0, b].at[idxv.at[c]], gxh.at[c], sem)
        pltpu.async_copy(cp_hbm.at[1, b].at[idxv.at[c]], gyh.at[c], sem)
        pltpu.async_copy(cp_hbm.at[2, b].at[idxv.at[c]], gzh.at[c], sem)
        pltpu.async_copy(vox_hbm.at[b].at[idxv.at[c]], gvh.at[c], sem)
        return 0

    lax.fori_loop(0, NCHUNK, chunk_fire, 0)

    def chunk_drain(c, acc):
        pltpu.make_async_copy(cp_hbm.at[0, b].at[idxv.at[c]], gxh.at[c], sem).wait()
        pltpu.make_async_copy(cp_hbm.at[1, b].at[idxv.at[c]], gyh.at[c], sem).wait()
        pltpu.make_async_copy(cp_hbm.at[2, b].at[idxv.at[c]], gzh.at[c], sem).wait()
        pltpu.make_async_copy(vox_hbm.at[b].at[idxv.at[c]], gvh.at[c], sem).wait()

        def phase_c(i, acc):
            off = pl.multiple_of(c * CHUNK + i * 16, 16)
            soff = pl.multiple_of(i * 16, 16)
            m = 1.0 - gvh[c, pl.ds(soff, 16)]
            dx = (rbx[pl.ds(off, 16)] - gxh[c, pl.ds(soff, 16)]) * m
            dy = (rby[pl.ds(off, 16)] - gyh[c, pl.ds(soff, 16)]) * m
            dz = (rbz[pl.ds(off, 16)] - gzh[c, pl.ds(soff, 16)]) * m
            return acc + (dx * dx + dy * dy + dz * dz)

        return lax.fori_loop(0, SUB, phase_c, acc)

    acc = lax.fori_loop(0, NCHUNK, chunk_drain, jnp.zeros((16,), jnp.float32))

    accv[pl.ds(0, 16)] = acc
    for r in range(1, 8):
        accv[pl.ds(r * 16, 16)] = zero16
    pltpu.sync_copy(accv, out_hbm.at[wid])


def _tc_reduce_body(x_ref, o_ref):
    o_ref[0, 0] = jnp.sum(x_ref[...]) * (1.0 / (B * N))


def kernel(voxel, points, closest_points, quads):
    pts_cm = points.transpose(2, 0, 1)
    cp_cm = closest_points.transpose(2, 0, 1)
    vox_2d = voxel.reshape(B, G3)
    quads_flat = quads.reshape(B * Q * 4)

    mesh = plsc.VectorSubcoreMesh(core_axis_name="c", subcore_axis_name="s")
    partials = pl.kernel(
        _sc_body,
        out_type=jax.ShapeDtypeStruct((NW, 128), jnp.float32),
        mesh=mesh,
        compiler_params=pltpu.CompilerParams(
            use_tc_tiling_on_sc=False, needs_layout_passes=False),
        scratch_types=[
            pltpu.VMEM((PPW,), jnp.float32),
            pltpu.VMEM((PPW,), jnp.float32),
            pltpu.VMEM((PPW,), jnp.float32),
            pltpu.VMEM((Q * 4,), jnp.float32),
            pltpu.VMEM((Q * 16,), jnp.float32),
            pltpu.VMEM((48,), jnp.float32),
            pltpu.VMEM((NCHUNK, CHUNK), jnp.int32),
            pltpu.VMEM((NCHUNK, CHUNK), jnp.float32),
            pltpu.VMEM((NCHUNK, CHUNK), jnp.float32),
            pltpu.VMEM((NCHUNK, CHUNK), jnp.float32),
            pltpu.VMEM((NCHUNK, CHUNK), jnp.float32),
            pltpu.VMEM((PPW,), jnp.float32),
            pltpu.VMEM((PPW,), jnp.float32),
            pltpu.VMEM((PPW,), jnp.float32),
            pltpu.VMEM((128,), jnp.float32),
            pltpu.VMEM_SHARED((16, 48), jnp.float32),
            pltpu.SemaphoreType.DMA,
        ],
    )(pts_cm, cp_cm, vox_2d, quads_flat)

    total = pl.pallas_call(
        _tc_reduce_body,
        out_shape=jax.ShapeDtypeStruct((1, 1), jnp.float32),
        out_specs=pl.BlockSpec(memory_space=pltpu.SMEM),
    )(partials)
    return total[0, 0]
